# Initial kernel scaffold; baseline (speedup 1.0000x reference)
import functools

import jax
import jax.numpy as jnp
from jax import lax
from jax.experimental import pallas as pl
from jax.experimental.pallas import tpu as pltpu
from jax.experimental.pallas import tpu_sc as plsc

N = 10000
E = 320000
G = 256
H = 128

NC = 2
NS = 16
NW = NC * NS
PER_W = E // NW
CH = 80
ROWS_PER_SUB = N // NS

_P = jax.lax.Precision.HIGHEST


def _silu(x):
    return x * jax.nn.sigmoid(x)


def _prep_body(nf_ref, wa_ref, wb_ref, p_ref, q_ref):
    x = nf_ref[...]
    p_ref[...] = jnp.dot(x, wa_ref[...], precision=_P)
    q_ref[...] = jnp.dot(x, wb_ref[...], precision=_P)


def _prep(nf, wa, wb):
    bn = 2500
    return pl.pallas_call(
        _prep_body,
        grid=(N // bn,),
        in_specs=[
            pl.BlockSpec((bn, H), lambda i: (i, 0)),
            pl.BlockSpec((H, H), lambda i: (0, 0)),
            pl.BlockSpec((H, H), lambda i: (0, 0)),
        ],
        out_specs=[
            pl.BlockSpec((bn, H), lambda i: (i, 0)),
            pl.BlockSpec((bn, H), lambda i: (i, 0)),
        ],
        out_shape=[
            jax.ShapeDtypeStruct((N, H), jnp.float32),
            jax.ShapeDtypeStruct((N, H), jnp.float32),
        ],
    )(nf, wa, wb)


def _lat_body(lat_ref, wc_ref, b_ref, l_ref):
    a = lat_ref[...]
    out = jnp.zeros((G, H), jnp.float32) + b_ref[...]
    for i in range(3):
        for j in range(3):
            ip = (a[:, 3 * i + 0:3 * i + 1] * a[:, 3 * j + 0:3 * j + 1]
                  + a[:, 3 * i + 1:3 * i + 2] * a[:, 3 * j + 1:3 * j + 2]
                  + a[:, 3 * i + 2:3 * i + 3] * a[:, 3 * j + 2:3 * j + 3])
            out = out + ip * wc_ref[3 * i + j:3 * i + j + 1, :]
    l_ref[...] = out


def _lat(lat9, wc, eb1):
    return pl.pallas_call(
        _lat_body,
        out_shape=jax.ShapeDtypeStruct((G, H), jnp.float32),
    )(lat9, wc, eb1)


def _sc_gather_body(p_hbm, q_hbm, l_hbm, src_hbm, dst_hbm, e2g_hbm,
                    hi_hbm, hj_hbm, lp_hbm,
                    i1, i2, i3, b1, b2, b3, s1, s2, s3):
    wid = lax.axis_index("s") * NC + lax.axis_index("c")
    base = wid * PER_W

    @pl.loop(0, PER_W, step=CH)
    def _(off):
        b = base + off
        pltpu.sync_copy(src_hbm.at[pl.ds(b, CH)], i1)
        pltpu.sync_copy(dst_hbm.at[pl.ds(b, CH)], i2)
        pltpu.sync_copy(e2g_hbm.at[pl.ds(b, CH)], i3)
        c1 = pltpu.async_copy(p_hbm.at[i1], b1, s1)
        c2 = pltpu.async_copy(q_hbm.at[i2], b2, s2)
        c3 = pltpu.async_copy(l_hbm.at[i3], b3, s3)
        c1.wait()
        c2.wait()
        c3.wait()
        pltpu.sync_copy(b1, hi_hbm.at[pl.ds(b, CH)])
        pltpu.sync_copy(b2, hj_hbm.at[pl.ds(b, CH)])
        pltpu.sync_copy(b3, lp_hbm.at[pl.ds(b, CH)])


def _sc_gather(p, q, lfull, src, dst, e2g):
    mesh = plsc.VectorSubcoreMesh(core_axis_name="c", subcore_axis_name="s")
    f = pl.kernel(
        _sc_gather_body,
        mesh=mesh,
        out_type=[jax.ShapeDtypeStruct((E, H), jnp.float32)] * 3,
        scratch_types=[
            pltpu.VMEM((CH,), jnp.int32),
            pltpu.VMEM((CH,), jnp.int32),
            pltpu.VMEM((CH,), jnp.int32),
            pltpu.VMEM((CH, H), jnp.float32),
            pltpu.VMEM((CH, H), jnp.float32),
            pltpu.VMEM((CH, H), jnp.float32),
            pltpu.SemaphoreType.DMA,
            pltpu.SemaphoreType.DMA,
            pltpu.SemaphoreType.DMA,
        ],
    )
    return f(p, q, lfull, src, dst, e2g)


def _edge_body(hi_ref, hj_ref, lp_ref, fd_ref, wd_ref, w2_ref, b2_ref, ef_ref):
    f = fd_ref[...]
    pre = hi_ref[...] + hj_ref[...] + lp_ref[...]
    for k in range(3):
        pre = pre + f[:, k:k + 1] * wd_ref[k:k + 1, :]
    h = _silu(pre)
    ef_ref[...] = _silu(jnp.dot(h, w2_ref[...], precision=_P) + b2_ref[...])


def _edge(hi, hj, lp, fd, wd, w2, b2):
    be = 2500
    return pl.pallas_call(
        _edge_body,
        grid=(E // be,),
        in_specs=[
            pl.BlockSpec((be, H), lambda i: (i, 0)),
            pl.BlockSpec((be, H), lambda i: (i, 0)),
            pl.BlockSpec((be, H), lambda i: (i, 0)),
            pl.BlockSpec((be, 3), lambda i: (i, 0)),
            pl.BlockSpec((3, H), lambda i: (0, 0)),
            pl.BlockSpec((H, H), lambda i: (0, 0)),
            pl.BlockSpec((1, H), lambda i: (0, 0)),
        ],
        out_specs=pl.BlockSpec((be, H), lambda i: (i, 0)),
        out_shape=jax.ShapeDtypeStruct((E, H), jnp.float32),
    )(hi, hj, lp, fd, wd, w2, b2)


def _sc_scatter_body(ef_hbm, dst_hbm, z_hbm, z16_hbm, ones_hbm,
                     sums_hbm, cnts_hbm,
                     efbuf, idxbuf, onesbuf, ssum, scnt):
    cid = lax.axis_index("c")
    sid = lax.axis_index("s")
    row0 = sid * ROWS_PER_SUB
    pltpu.sync_copy(z_hbm.at[pl.ds(row0, ROWS_PER_SUB)],
                    ssum.at[pl.ds(row0, ROWS_PER_SUB)])
    pltpu.sync_copy(z16_hbm.at[pl.ds(row0, ROWS_PER_SUB)],
                    scnt.at[pl.ds(row0, ROWS_PER_SUB)])
    pltpu.sync_copy(ones_hbm, onesbuf)
    plsc.subcore_barrier()

    wid = sid * NC + cid
    base = wid * PER_W

    @pl.loop(0, PER_W, step=CH)
    def _(off):
        b = base + off
        pltpu.sync_copy(dst_hbm.at[pl.ds(b, CH)], idxbuf)
        pltpu.sync_copy(ef_hbm.at[pl.ds(b, CH)], efbuf)
        pltpu.sync_copy(efbuf, ssum.at[idxbuf], add=True)
        pltpu.sync_copy(onesbuf, scnt.at[idxbuf], add=True)

    plsc.subcore_barrier()
    pltpu.sync_copy(ssum.at[pl.ds(row0, ROWS_PER_SUB)],
                    sums_hbm.at[cid, pl.ds(row0, ROWS_PER_SUB)])
    pltpu.sync_copy(scnt.at[pl.ds(row0, ROWS_PER_SUB)],
                    cnts_hbm.at[cid, pl.ds(row0, ROWS_PER_SUB)])


def _sc_scatter(ef, dst, zeros_n, zeros16, ones16):
    mesh = plsc.VectorSubcoreMesh(core_axis_name="c", subcore_axis_name="s")
    f = pl.kernel(
        _sc_scatter_body,
        mesh=mesh,
        out_type=[
            jax.ShapeDtypeStruct((NC, N, H), jnp.float32),
            jax.ShapeDtypeStruct((NC, N, 16), jnp.float32),
        ],
        scratch_types=[
            pltpu.VMEM((CH, H), jnp.float32),
            pltpu.VMEM((CH,), jnp.int32),
            pltpu.VMEM((CH, 16), jnp.float32),
            pltpu.VMEM_SHARED((N, H), jnp.float32),
            pltpu.VMEM_SHARED((N, 16), jnp.float32),
        ],
    )
    return f(ef, dst, zeros_n, zeros16, ones16)


def _node_body(nf_ref, sp_ref, cp_ref, wa_ref, wb_ref, b1_ref, w2_ref, b2_ref,
               y_ref):
    nf = nf_ref[...]
    s = sp_ref[0] + sp_ref[1]
    c = cp_ref[0, :, 0:1] + cp_ref[1, :, 0:1]
    agg = s / jnp.maximum(c, 1.0)
    o1 = _silu(jnp.dot(nf, wa_ref[...], precision=_P)
               + jnp.dot(agg, wb_ref[...], precision=_P) + b1_ref[...])
    y_ref[...] = nf + _silu(jnp.dot(o1, w2_ref[...], precision=_P) + b2_ref[...])


def _node(nf, sums, cnts, nw1a, nw1b, nb1, nw2, nb2):
    bn = 2500
    return pl.pallas_call(
        _node_body,
        grid=(N // bn,),
        in_specs=[
            pl.BlockSpec((bn, H), lambda i: (i, 0)),
            pl.BlockSpec((NC, bn, H), lambda i: (0, i, 0)),
            pl.BlockSpec((NC, bn, 16), lambda i: (0, i, 0)),
            pl.BlockSpec((H, H), lambda i: (0, 0)),
            pl.BlockSpec((H, H), lambda i: (0, 0)),
            pl.BlockSpec((1, H), lambda i: (0, 0)),
            pl.BlockSpec((H, H), lambda i: (0, 0)),
            pl.BlockSpec((1, H), lambda i: (0, 0)),
        ],
        out_specs=pl.BlockSpec((bn, H), lambda i: (i, 0)),
        out_shape=jax.ShapeDtypeStruct((N, H), jnp.float32),
    )(nf, sums, cnts, nw1a, nw1b, nb1, nw2, nb2)


def kernel(node_features, frac_coords, lattices, edge_index, edge2graph,
           frac_diff, ew1, eb1, ew2, eb2, nw1, nb1, nw2, nb2):
    del frac_coords
    lat9 = lattices.reshape(G, 9)
    src = edge_index[0]
    dst = edge_index[1]

    p, q = _prep(node_features, ew1[:H], ew1[H:2 * H])
    lfull = _lat(lat9, ew1[2 * H:2 * H + 9], eb1.reshape(1, H))
    hi, hj, lp = _sc_gather(p, q, lfull, src, dst, edge2graph)
    ef = _edge(hi, hj, lp, frac_diff, ew1[2 * H + 9:], ew2, eb2.reshape(1, H))
    sums, cnts = _sc_scatter(
        ef, dst,
        jnp.zeros((N, H), jnp.float32),
        jnp.zeros((N, 16), jnp.float32),
        jnp.ones((CH, 16), jnp.float32),
    )
    y = _node(node_features, sums, cnts, nw1[:H], nw1[H:], nb1.reshape(1, H),
              nw2, nb2.reshape(1, H))
    return (y, ef)


# speedup vs baseline: 1.4621x; 1.4621x over previous
import dataclasses
import functools

import jax
import jax.numpy as jnp
from jax import lax
from jax.experimental import pallas as pl
from jax.experimental.pallas import tpu as pltpu
from jax.experimental.pallas import tpu_sc as plsc

N = 10000
E = 320000
G = 256
H = 128

NC = 2
NS = 16
NW = NC * NS
PER_W = E // NW
PER_S = E // NS
CH = 80
ROWS_A = 624
TAIL0 = ROWS_A * NS
TAIL = N - TAIL0

_P = jax.lax.Precision.HIGHEST


def _silu(x):
    return x * jax.nn.sigmoid(x)


def _prep_body(nf_ref, wa_ref, wb_ref, p_ref, q_ref):
    x = nf_ref[...]
    p_ref[...] = jnp.dot(x, wa_ref[...], precision=_P)
    q_ref[...] = jnp.dot(x, wb_ref[...], precision=_P)


def _prep(nf, wa, wb):
    bn = 2000
    return pl.pallas_call(
        _prep_body,
        grid=(N // bn,),
        in_specs=[
            pl.BlockSpec((bn, H), lambda i: (i, 0)),
            pl.BlockSpec((H, H), lambda i: (0, 0)),
            pl.BlockSpec((H, H), lambda i: (0, 0)),
        ],
        out_specs=[
            pl.BlockSpec((bn, H), lambda i: (i, 0)),
            pl.BlockSpec((bn, H), lambda i: (i, 0)),
        ],
        out_shape=[
            jax.ShapeDtypeStruct((N, H), jnp.float32),
            jax.ShapeDtypeStruct((N, H), jnp.float32),
        ],
    )(nf, wa, wb)


def _lat_body(lat_ref, wc_ref, b_ref, l_ref):
    a = lat_ref[...]
    out = jnp.zeros((G, H), jnp.float32) + b_ref[...]
    for i in range(3):
        for j in range(3):
            ip = (a[:, 3 * i + 0:3 * i + 1] * a[:, 3 * j + 0:3 * j + 1]
                  + a[:, 3 * i + 1:3 * i + 2] * a[:, 3 * j + 1:3 * j + 2]
                  + a[:, 3 * i + 2:3 * i + 3] * a[:, 3 * j + 2:3 * j + 3])
            out = out + ip * wc_ref[3 * i + j:3 * i + j + 1, :]
    l_ref[...] = out


def _lat(lat9, wc, eb1):
    return pl.pallas_call(
        _lat_body,
        out_shape=jax.ShapeDtypeStruct((G, H), jnp.float32),
    )(lat9, wc, eb1)


def _sc_gather_body(p_hbm, q_hbm, l_hbm, src_hbm, dst_hbm, e2g_hbm,
                    hi_hbm, hj_hbm, lp_hbm,
                    i1, i2, i3, b1, b2, b3, s1, s2, s3):
    wid = lax.axis_index("s") * NC + lax.axis_index("c")
    base = wid * PER_W

    @pl.loop(0, PER_W, step=CH)
    def _(off):
        b = base + off
        pltpu.sync_copy(src_hbm.at[pl.ds(b, CH)], i1)
        pltpu.sync_copy(dst_hbm.at[pl.ds(b, CH)], i2)
        pltpu.sync_copy(e2g_hbm.at[pl.ds(b, CH)], i3)
        c1 = pltpu.async_copy(p_hbm.at[i1], b1, s1)
        c2 = pltpu.async_copy(q_hbm.at[i2], b2, s2)
        c3 = pltpu.async_copy(l_hbm.at[i3], b3, s3)
        c1.wait()
        c2.wait()
        c3.wait()
        pltpu.sync_copy(b1, hi_hbm.at[pl.ds(b, CH)])
        pltpu.sync_copy(b2, hj_hbm.at[pl.ds(b, CH)])
        pltpu.sync_copy(b3, lp_hbm.at[pl.ds(b, CH)])


def _sc_gather(p, q, lfull, src, dst, e2g):
    mesh = plsc.VectorSubcoreMesh(core_axis_name="c", subcore_axis_name="s")
    f = pl.kernel(
        _sc_gather_body,
        mesh=mesh,
        out_type=[jax.ShapeDtypeStruct((E, H), jnp.float32)] * 3,
        scratch_types=[
            pltpu.VMEM((CH,), jnp.int32),
            pltpu.VMEM((CH,), jnp.int32),
            pltpu.VMEM((CH,), jnp.int32),
            pltpu.VMEM((CH, H), jnp.float32),
            pltpu.VMEM((CH, H), jnp.float32),
            pltpu.VMEM((CH, H), jnp.float32),
            pltpu.SemaphoreType.DMA,
            pltpu.SemaphoreType.DMA,
            pltpu.SemaphoreType.DMA,
        ],
    )
    return f(p, q, lfull, src, dst, e2g)


def _edge_body(hi_ref, hj_ref, lp_ref, fd_ref, wd_ref, w2_ref, b2_ref, ef_ref):
    f = fd_ref[...]
    pre = hi_ref[...] + hj_ref[...] + lp_ref[...]
    for k in range(3):
        pre = pre + f[:, k:k + 1] * wd_ref[k:k + 1, :]
    h = _silu(pre)
    ef_ref[...] = _silu(jnp.dot(h, w2_ref[...], precision=_P) + b2_ref[...])


def _edge(hi, hj, lp, fd, wd, w2, b2):
    be = 2000
    return pl.pallas_call(
        _edge_body,
        grid=(E // be,),
        in_specs=[
            pl.BlockSpec((be, H), lambda i: (i, 0)),
            pl.BlockSpec((be, H), lambda i: (i, 0)),
            pl.BlockSpec((be, H), lambda i: (i, 0)),
            pl.BlockSpec((be, 3), lambda i: (i, 0)),
            pl.BlockSpec((3, H), lambda i: (0, 0)),
            pl.BlockSpec((H, H), lambda i: (0, 0)),
            pl.BlockSpec((1, H), lambda i: (0, 0)),
        ],
        out_specs=pl.BlockSpec((be, H), lambda i: (i, 0)),
        out_shape=jax.ShapeDtypeStruct((E, H), jnp.float32),
    )(hi, hj, lp, fd, wd, w2, b2)


def _sc_scatter_body(ef_hbm, dst_hbm, z_hbm, ones_hbm,
                     sums_hbm, cntb_hbm,
                     efbuf, idxbuf, onesbuf, ssum):
    cid = lax.axis_index("c")
    sid = lax.axis_index("s")
    row0 = sid * ROWS_A
    pltpu.sync_copy(z_hbm.at[pl.ds(row0, ROWS_A)],
                    ssum.at[pl.ds(row0, ROWS_A)])

    @pl.when(sid == 0)
    def _():
        pltpu.sync_copy(z_hbm.at[pl.ds(TAIL0, TAIL)],
                        ssum.at[pl.ds(TAIL0, TAIL)])

    pltpu.sync_copy(ones_hbm, onesbuf)
    plsc.subcore_barrier()

    base = sid * PER_S

    @pl.loop(0, PER_S, step=CH)
    def _(off):
        b = base + off
        pltpu.sync_copy(dst_hbm.at[pl.ds(b, CH)], idxbuf)

        @pl.when(cid == 0)
        def _():
            pltpu.sync_copy(ef_hbm.at[pl.ds(b, CH)], efbuf)
            pltpu.sync_copy(efbuf, ssum.at[idxbuf], add=True)

        @pl.when(cid == 1)
        def _():
            pltpu.sync_copy(onesbuf, ssum.at[idxbuf], add=True)

    plsc.subcore_barrier()

    @pl.when(cid == 0)
    def _():
        pltpu.sync_copy(ssum.at[pl.ds(row0, ROWS_A)],
                        sums_hbm.at[pl.ds(row0, ROWS_A)])

        @pl.when(sid == 0)
        def _():
            pltpu.sync_copy(ssum.at[pl.ds(TAIL0, TAIL)],
                            sums_hbm.at[pl.ds(TAIL0, TAIL)])

    @pl.when(cid == 1)
    def _():
        pltpu.sync_copy(ssum.at[pl.ds(row0, ROWS_A)],
                        cntb_hbm.at[pl.ds(row0, ROWS_A)])

        @pl.when(sid == 0)
        def _():
            pltpu.sync_copy(ssum.at[pl.ds(TAIL0, TAIL)],
                            cntb_hbm.at[pl.ds(TAIL0, TAIL)])


def _sc_scatter(ef, dst, zeros_n, ones_ch):
    mesh = plsc.VectorSubcoreMesh(core_axis_name="c", subcore_axis_name="s")
    f = pl.kernel(
        _sc_scatter_body,
        mesh=mesh,
        out_type=[
            jax.ShapeDtypeStruct((N, H), jnp.float32),
            jax.ShapeDtypeStruct((N, H), jnp.float32),
        ],
        scratch_types=[
            pltpu.VMEM((CH, H), jnp.float32),
            pltpu.VMEM((CH,), jnp.int32),
            pltpu.VMEM((CH, H), jnp.float32),
            pltpu.VMEM_SHARED((N, H), jnp.float32),
        ],
    )
    return f(ef, dst, zeros_n, ones_ch)


def _node_body(nf_ref, sp_ref, cp_ref, wa_ref, wb_ref, b1_ref, w2_ref, b2_ref,
               y_ref):
    nf = nf_ref[...]
    s = sp_ref[...]
    c = cp_ref[:, 0:1]
    agg = s / jnp.maximum(c, 1.0)
    o1 = _silu(jnp.dot(nf, wa_ref[...], precision=_P)
               + jnp.dot(agg, wb_ref[...], precision=_P) + b1_ref[...])
    y_ref[...] = nf + _silu(jnp.dot(o1, w2_ref[...], precision=_P) + b2_ref[...])


def _node(nf, sums, cnts, nw1a, nw1b, nb1, nw2, nb2):
    bn = 2000
    return pl.pallas_call(
        _node_body,
        grid=(N // bn,),
        in_specs=[
            pl.BlockSpec((bn, H), lambda i: (i, 0)),
            pl.BlockSpec((bn, H), lambda i: (i, 0)),
            pl.BlockSpec((bn, H), lambda i: (i, 0)),
            pl.BlockSpec((H, H), lambda i: (0, 0)),
            pl.BlockSpec((H, H), lambda i: (0, 0)),
            pl.BlockSpec((1, H), lambda i: (0, 0)),
            pl.BlockSpec((H, H), lambda i: (0, 0)),
            pl.BlockSpec((1, H), lambda i: (0, 0)),
        ],
        out_specs=pl.BlockSpec((bn, H), lambda i: (i, 0)),
        out_shape=jax.ShapeDtypeStruct((N, H), jnp.float32),
    )(nf, sums, cnts, nw1a, nw1b, nb1, nw2, nb2)


def kernel(node_features, frac_coords, lattices, edge_index, edge2graph,
           frac_diff, ew1, eb1, ew2, eb2, nw1, nb1, nw2, nb2):
    del frac_coords
    lat9 = lattices.reshape(G, 9)
    src = edge_index[0]
    dst = edge_index[1]

    p, q = _prep(node_features, ew1[:H], ew1[H:2 * H])
    lfull = _lat(lat9, ew1[2 * H:2 * H + 9], eb1.reshape(1, H))
    hi, hj, lp = _sc_gather(p, q, lfull, src, dst, edge2graph)
    ef = _edge(hi, hj, lp, frac_diff, ew1[2 * H + 9:], ew2, eb2.reshape(1, H))
    sums, cnts = _sc_scatter(ef, dst,
                             jnp.zeros((N, H), jnp.float32),
                             jnp.ones((CH, H), jnp.float32))
    y = _node(node_features, sums, cnts, nw1[:H], nw1[H:], nb1.reshape(1, H),
              nw2, nb2.reshape(1, H))
    return (y, ef)


# speedup vs baseline: 1.5911x; 1.0882x over previous
import dataclasses
import functools

import jax
import jax.numpy as jnp
from jax import lax
from jax.experimental import pallas as pl
from jax.experimental.pallas import tpu as pltpu
from jax.experimental.pallas import tpu_sc as plsc

N = 10000
E = 320000
G = 256
H = 128

NC = 2
NS = 16
NW = NC * NS
PER_W = E // NW
PER_S = E // NS
CH = 80
ROWS_A = 624
TAIL0 = ROWS_A * NS
TAIL = N - TAIL0

_P = jax.lax.Precision.HIGHEST


def _silu(x):
    return x * jax.nn.sigmoid(x)


def _prep_body(nf_ref, wa_ref, wb_ref, p_ref, q_ref):
    x = nf_ref[...]
    p_ref[...] = jnp.dot(x, wa_ref[...], precision=_P)
    q_ref[...] = jnp.dot(x, wb_ref[...], precision=_P)


def _prep(nf, wa, wb):
    bn = 2000
    return pl.pallas_call(
        _prep_body,
        grid=(N // bn,),
        in_specs=[
            pl.BlockSpec((bn, H), lambda i: (i, 0)),
            pl.BlockSpec((H, H), lambda i: (0, 0)),
            pl.BlockSpec((H, H), lambda i: (0, 0)),
        ],
        out_specs=[
            pl.BlockSpec((bn, H), lambda i: (i, 0)),
            pl.BlockSpec((bn, H), lambda i: (i, 0)),
        ],
        out_shape=[
            jax.ShapeDtypeStruct((N, H), jnp.float32),
            jax.ShapeDtypeStruct((N, H), jnp.float32),
        ],
    )(nf, wa, wb)


def _lat_body(lat_ref, wc_ref, b_ref, l_ref):
    a = lat_ref[...]
    out = jnp.zeros((G, H), jnp.float32) + b_ref[...]
    for i in range(3):
        for j in range(3):
            ip = (a[:, 3 * i + 0:3 * i + 1] * a[:, 3 * j + 0:3 * j + 1]
                  + a[:, 3 * i + 1:3 * i + 2] * a[:, 3 * j + 1:3 * j + 2]
                  + a[:, 3 * i + 2:3 * i + 3] * a[:, 3 * j + 2:3 * j + 3])
            out = out + ip * wc_ref[3 * i + j:3 * i + j + 1, :]
    l_ref[...] = out


def _lat(lat9, wc, eb1):
    return pl.pallas_call(
        _lat_body,
        out_shape=jax.ShapeDtypeStruct((G, H), jnp.float32),
    )(lat9, wc, eb1)


CHG = 96
NCHG = PER_W // CHG
GTAIL0 = NCHG * CHG
GTAIL = PER_W - GTAIL0


def _sc_gather_body(p_hbm, q_hbm, l_hbm, src_hbm, dst_hbm, e2g_hbm,
                    hi_hbm, hj_hbm, lp_hbm,
                    isl1, isl2, isl3,
                    gb1a, gb2a, gb3a, gb1b, gb2b, gb3b,
                    sga, sgb, swa, swb):
    wid = lax.axis_index("s") * NC + lax.axis_index("c")
    base = wid * PER_W

    pltpu.sync_copy(src_hbm.at[pl.ds(base, PER_W)], isl1)
    pltpu.sync_copy(dst_hbm.at[pl.ds(base, PER_W)], isl2)
    pltpu.sync_copy(e2g_hbm.at[pl.ds(base, PER_W)], isl3)

    bufs = ((gb1a, gb2a, gb3a), (gb1b, gb2b, gb3b))
    gsem = (sga, sgb)
    wsem = (swa, swb)

    def fire_gather(c, s):
        off = c * CHG
        b1, b2, b3 = bufs[s]
        pltpu.async_copy(p_hbm.at[isl1.at[pl.ds(off, CHG)]], b1, gsem[s])
        pltpu.async_copy(q_hbm.at[isl2.at[pl.ds(off, CHG)]], b2, gsem[s])
        pltpu.async_copy(l_hbm.at[isl3.at[pl.ds(off, CHG)]], b3, gsem[s])

    def drain_gather(c, s):
        off = c * CHG
        b1, b2, b3 = bufs[s]
        pltpu.make_async_copy(p_hbm.at[isl1.at[pl.ds(off, CHG)]], b1, gsem[s]).wait()
        pltpu.make_async_copy(q_hbm.at[isl2.at[pl.ds(off, CHG)]], b2, gsem[s]).wait()
        pltpu.make_async_copy(l_hbm.at[isl3.at[pl.ds(off, CHG)]], b3, gsem[s]).wait()

    def fire_wb(c, s):
        g = base + c * CHG
        b1, b2, b3 = bufs[s]
        pltpu.async_copy(b1, hi_hbm.at[pl.ds(g, CHG)], wsem[s])
        pltpu.async_copy(b2, hj_hbm.at[pl.ds(g, CHG)], wsem[s])
        pltpu.async_copy(b3, lp_hbm.at[pl.ds(g, CHG)], wsem[s])

    def drain_wb(c, s):
        g = base + c * CHG
        b1, b2, b3 = bufs[s]
        pltpu.make_async_copy(b1, hi_hbm.at[pl.ds(g, CHG)], wsem[s]).wait()
        pltpu.make_async_copy(b2, hj_hbm.at[pl.ds(g, CHG)], wsem[s]).wait()
        pltpu.make_async_copy(b3, lp_hbm.at[pl.ds(g, CHG)], wsem[s]).wait()

    def block(c, s):
        drain_gather(c, s)

        @pl.when(c >= 1)
        def _():
            drain_wb(c - 1, 1 - s)

        @pl.when(c + 1 < NCHG)
        def _():
            fire_gather(c + 1, 1 - s)

        fire_wb(c, s)

    fire_gather(0, 0)

    @pl.loop(0, NCHG)
    def _(c):
        @pl.when(c % 2 == 0)
        def _():
            block(c, 0)

        @pl.when(c % 2 == 1)
        def _():
            block(c, 1)

    drain_wb(NCHG - 1, (NCHG - 1) % 2)

    b1, b2, b3 = bufs[0]
    t = GTAIL0
    g = base + t
    pltpu.async_copy(p_hbm.at[isl1.at[pl.ds(t, GTAIL)]], b1.at[pl.ds(0, GTAIL)], sga).wait()
    pltpu.async_copy(q_hbm.at[isl2.at[pl.ds(t, GTAIL)]], b2.at[pl.ds(0, GTAIL)], sga).wait()
    pltpu.async_copy(l_hbm.at[isl3.at[pl.ds(t, GTAIL)]], b3.at[pl.ds(0, GTAIL)], sga).wait()
    pltpu.sync_copy(b1.at[pl.ds(0, GTAIL)], hi_hbm.at[pl.ds(g, GTAIL)])
    pltpu.sync_copy(b2.at[pl.ds(0, GTAIL)], hj_hbm.at[pl.ds(g, GTAIL)])
    pltpu.sync_copy(b3.at[pl.ds(0, GTAIL)], lp_hbm.at[pl.ds(g, GTAIL)])


def _sc_gather(p, q, lfull, src, dst, e2g):
    mesh = plsc.VectorSubcoreMesh(core_axis_name="c", subcore_axis_name="s")
    f = pl.kernel(
        _sc_gather_body,
        mesh=mesh,
        out_type=[jax.ShapeDtypeStruct((E, H), jnp.float32)] * 3,
        scratch_types=[
            pltpu.VMEM((PER_W,), jnp.int32),
            pltpu.VMEM((PER_W,), jnp.int32),
            pltpu.VMEM((PER_W,), jnp.int32),
            pltpu.VMEM((CHG, H), jnp.float32),
            pltpu.VMEM((CHG, H), jnp.float32),
            pltpu.VMEM((CHG, H), jnp.float32),
            pltpu.VMEM((CHG, H), jnp.float32),
            pltpu.VMEM((CHG, H), jnp.float32),
            pltpu.VMEM((CHG, H), jnp.float32),
            pltpu.SemaphoreType.DMA,
            pltpu.SemaphoreType.DMA,
            pltpu.SemaphoreType.DMA,
            pltpu.SemaphoreType.DMA,
        ],
    )
    return f(p, q, lfull, src, dst, e2g)


def _edge_body(hi_ref, hj_ref, lp_ref, fd_ref, wd_ref, w2_ref, b2_ref, ef_ref):
    f = fd_ref[...]
    pre = hi_ref[...] + hj_ref[...] + lp_ref[...]
    for k in range(3):
        pre = pre + f[:, k:k + 1] * wd_ref[k:k + 1, :]
    h = _silu(pre)
    ef_ref[...] = _silu(jnp.dot(h, w2_ref[...], precision=_P) + b2_ref[...])


def _edge(hi, hj, lp, fd, wd, w2, b2):
    be = 2000
    return pl.pallas_call(
        _edge_body,
        grid=(E // be,),
        in_specs=[
            pl.BlockSpec((be, H), lambda i: (i, 0)),
            pl.BlockSpec((be, H), lambda i: (i, 0)),
            pl.BlockSpec((be, H), lambda i: (i, 0)),
            pl.BlockSpec((be, 3), lambda i: (i, 0)),
            pl.BlockSpec((3, H), lambda i: (0, 0)),
            pl.BlockSpec((H, H), lambda i: (0, 0)),
            pl.BlockSpec((1, H), lambda i: (0, 0)),
        ],
        out_specs=pl.BlockSpec((be, H), lambda i: (i, 0)),
        out_shape=jax.ShapeDtypeStruct((E, H), jnp.float32),
    )(hi, hj, lp, fd, wd, w2, b2)


def _sc_scatter_body(ef_hbm, dst_hbm, z_hbm, ones_hbm,
                     sums_hbm, cntb_hbm,
                     efbuf, idxbuf, onesbuf, ssum):
    cid = lax.axis_index("c")
    sid = lax.axis_index("s")
    row0 = sid * ROWS_A
    pltpu.sync_copy(z_hbm.at[pl.ds(row0, ROWS_A)],
                    ssum.at[pl.ds(row0, ROWS_A)])

    @pl.when(sid == 0)
    def _():
        pltpu.sync_copy(z_hbm.at[pl.ds(TAIL0, TAIL)],
                        ssum.at[pl.ds(TAIL0, TAIL)])

    pltpu.sync_copy(ones_hbm, onesbuf)
    plsc.subcore_barrier()

    base = sid * PER_S

    @pl.loop(0, PER_S, step=CH)
    def _(off):
        b = base + off
        pltpu.sync_copy(dst_hbm.at[pl.ds(b, CH)], idxbuf)

        @pl.when(cid == 0)
        def _():
            pltpu.sync_copy(ef_hbm.at[pl.ds(b, CH)], efbuf)
            pltpu.sync_copy(efbuf, ssum.at[idxbuf], add=True)

        @pl.when(cid == 1)
        def _():
            pltpu.sync_copy(onesbuf, ssum.at[idxbuf], add=True)

    plsc.subcore_barrier()

    @pl.when(cid == 0)
    def _():
        pltpu.sync_copy(ssum.at[pl.ds(row0, ROWS_A)],
                        sums_hbm.at[pl.ds(row0, ROWS_A)])

        @pl.when(sid == 0)
        def _():
            pltpu.sync_copy(ssum.at[pl.ds(TAIL0, TAIL)],
                            sums_hbm.at[pl.ds(TAIL0, TAIL)])

    @pl.when(cid == 1)
    def _():
        pltpu.sync_copy(ssum.at[pl.ds(row0, ROWS_A)],
                        cntb_hbm.at[pl.ds(row0, ROWS_A)])

        @pl.when(sid == 0)
        def _():
            pltpu.sync_copy(ssum.at[pl.ds(TAIL0, TAIL)],
                            cntb_hbm.at[pl.ds(TAIL0, TAIL)])


def _sc_scatter(ef, dst, zeros_n, ones_ch):
    mesh = plsc.VectorSubcoreMesh(core_axis_name="c", subcore_axis_name="s")
    f = pl.kernel(
        _sc_scatter_body,
        mesh=mesh,
        out_type=[
            jax.ShapeDtypeStruct((N, H), jnp.float32),
            jax.ShapeDtypeStruct((N, H), jnp.float32),
        ],
        scratch_types=[
            pltpu.VMEM((CH, H), jnp.float32),
            pltpu.VMEM((CH,), jnp.int32),
            pltpu.VMEM((CH, H), jnp.float32),
            pltpu.VMEM_SHARED((N, H), jnp.float32),
        ],
    )
    return f(ef, dst, zeros_n, ones_ch)


def _node_body(nf_ref, sp_ref, cp_ref, wa_ref, wb_ref, b1_ref, w2_ref, b2_ref,
               y_ref):
    nf = nf_ref[...]
    s = sp_ref[...]
    c = cp_ref[:, 0:1]
    agg = s / jnp.maximum(c, 1.0)
    o1 = _silu(jnp.dot(nf, wa_ref[...], precision=_P)
               + jnp.dot(agg, wb_ref[...], precision=_P) + b1_ref[...])
    y_ref[...] = nf + _silu(jnp.dot(o1, w2_ref[...], precision=_P) + b2_ref[...])


def _node(nf, sums, cnts, nw1a, nw1b, nb1, nw2, nb2):
    bn = 2000
    return pl.pallas_call(
        _node_body,
        grid=(N // bn,),
        in_specs=[
            pl.BlockSpec((bn, H), lambda i: (i, 0)),
            pl.BlockSpec((bn, H), lambda i: (i, 0)),
            pl.BlockSpec((bn, H), lambda i: (i, 0)),
            pl.BlockSpec((H, H), lambda i: (0, 0)),
            pl.BlockSpec((H, H), lambda i: (0, 0)),
            pl.BlockSpec((1, H), lambda i: (0, 0)),
            pl.BlockSpec((H, H), lambda i: (0, 0)),
            pl.BlockSpec((1, H), lambda i: (0, 0)),
        ],
        out_specs=pl.BlockSpec((bn, H), lambda i: (i, 0)),
        out_shape=jax.ShapeDtypeStruct((N, H), jnp.float32),
    )(nf, sums, cnts, nw1a, nw1b, nb1, nw2, nb2)


def kernel(node_features, frac_coords, lattices, edge_index, edge2graph,
           frac_diff, ew1, eb1, ew2, eb2, nw1, nb1, nw2, nb2):
    del frac_coords
    lat9 = lattices.reshape(G, 9)
    src = edge_index[0]
    dst = edge_index[1]

    p, q = _prep(node_features, ew1[:H], ew1[H:2 * H])
    lfull = _lat(lat9, ew1[2 * H:2 * H + 9], eb1.reshape(1, H))
    hi, hj, lp = _sc_gather(p, q, lfull, src, dst, edge2graph)
    ef = _edge(hi, hj, lp, frac_diff, ew1[2 * H + 9:], ew2, eb2.reshape(1, H))
    sums, cnts = _sc_scatter(ef, dst,
                             jnp.zeros((N, H), jnp.float32),
                             jnp.ones((CH, H), jnp.float32))
    y = _node(node_features, sums, cnts, nw1[:H], nw1[H:], nb1.reshape(1, H),
              nw2, nb2.reshape(1, H))
    return (y, ef)


# speedup vs baseline: 2.1897x; 1.3763x over previous
import dataclasses
import functools

import jax
import jax.numpy as jnp
from jax import lax
from jax.experimental import pallas as pl
from jax.experimental.pallas import tpu as pltpu
from jax.experimental.pallas import tpu_sc as plsc

N = 10000
E = 320000
G = 256
H = 128

NC = 2
NS = 16
NW = NC * NS
PER_W = E // NW
PER_S = E // NS
CH = 80
ROWS_A = 624
TAIL0 = ROWS_A * NS
TAIL = N - TAIL0

_P = jax.lax.Precision.HIGHEST


def _silu(x):
    return x * jax.nn.sigmoid(x)


def _prep_body(nf_ref, wa_ref, wb_ref, p_ref, q_ref):
    x = nf_ref[...]
    p_ref[...] = jnp.dot(x, wa_ref[...], precision=_P)
    q_ref[...] = jnp.dot(x, wb_ref[...], precision=_P)


def _prep(nf, wa, wb):
    bn = 2000
    return pl.pallas_call(
        _prep_body,
        grid=(N // bn,),
        in_specs=[
            pl.BlockSpec((bn, H), lambda i: (i, 0)),
            pl.BlockSpec((H, H), lambda i: (0, 0)),
            pl.BlockSpec((H, H), lambda i: (0, 0)),
        ],
        out_specs=[
            pl.BlockSpec((bn, H), lambda i: (i, 0)),
            pl.BlockSpec((bn, H), lambda i: (i, 0)),
        ],
        out_shape=[
            jax.ShapeDtypeStruct((N, H), jnp.float32),
            jax.ShapeDtypeStruct((N, H), jnp.float32),
        ],
    )(nf, wa, wb)


def _lat_body(lat_ref, wc_ref, b_ref, l_ref):
    a = lat_ref[...]
    out = jnp.zeros((G, H), jnp.float32) + b_ref[...]
    for i in range(3):
        for j in range(3):
            ip = (a[:, 3 * i + 0:3 * i + 1] * a[:, 3 * j + 0:3 * j + 1]
                  + a[:, 3 * i + 1:3 * i + 2] * a[:, 3 * j + 1:3 * j + 2]
                  + a[:, 3 * i + 2:3 * i + 3] * a[:, 3 * j + 2:3 * j + 3])
            out = out + ip * wc_ref[3 * i + j:3 * i + j + 1, :]
    l_ref[...] = out


def _lat(lat9, wc, eb1):
    return pl.pallas_call(
        _lat_body,
        out_shape=jax.ShapeDtypeStruct((G, H), jnp.float32),
    )(lat9, wc, eb1)


CHG = 128
NCHG = PER_W // CHG
GTAIL0 = NCHG * CHG
GTAIL = PER_W - GTAIL0


def _sc_gather_body(p_hbm, q_hbm, src_hbm, dst_hbm,
                    hi_hbm, hj_hbm,
                    isl1, isl2,
                    gb1a, gb2a, gb1b, gb2b,
                    sga, sgb, swa, swb):
    wid = lax.axis_index("s") * NC + lax.axis_index("c")
    base = wid * PER_W

    pltpu.sync_copy(src_hbm.at[pl.ds(base, PER_W)], isl1)
    pltpu.sync_copy(dst_hbm.at[pl.ds(base, PER_W)], isl2)

    bufs = ((gb1a, gb2a), (gb1b, gb2b))
    gsem = (sga, sgb)
    wsem = (swa, swb)

    def fire_gather(c, s):
        off = c * CHG
        b1, b2 = bufs[s]
        pltpu.async_copy(p_hbm.at[isl1.at[pl.ds(off, CHG)]], b1, gsem[s])
        pltpu.async_copy(q_hbm.at[isl2.at[pl.ds(off, CHG)]], b2, gsem[s])

    def drain_gather(c, s):
        off = c * CHG
        b1, b2 = bufs[s]
        pltpu.make_async_copy(p_hbm.at[isl1.at[pl.ds(off, CHG)]], b1, gsem[s]).wait()
        pltpu.make_async_copy(q_hbm.at[isl2.at[pl.ds(off, CHG)]], b2, gsem[s]).wait()

    def fire_wb(c, s):
        g = base + c * CHG
        b1, b2 = bufs[s]
        pltpu.async_copy(b1, hi_hbm.at[pl.ds(g, CHG)], wsem[s])
        pltpu.async_copy(b2, hj_hbm.at[pl.ds(g, CHG)], wsem[s])

    def drain_wb(c, s):
        g = base + c * CHG
        b1, b2 = bufs[s]
        pltpu.make_async_copy(b1, hi_hbm.at[pl.ds(g, CHG)], wsem[s]).wait()
        pltpu.make_async_copy(b2, hj_hbm.at[pl.ds(g, CHG)], wsem[s]).wait()

    def block(c, s):
        drain_gather(c, s)

        @pl.when(c >= 1)
        def _():
            drain_wb(c - 1, 1 - s)

        @pl.when(c + 1 < NCHG)
        def _():
            fire_gather(c + 1, 1 - s)

        fire_wb(c, s)

    fire_gather(0, 0)

    @pl.loop(0, NCHG)
    def _(c):
        @pl.when(c % 2 == 0)
        def _():
            block(c, 0)

        @pl.when(c % 2 == 1)
        def _():
            block(c, 1)

    drain_wb(NCHG - 1, (NCHG - 1) % 2)

    b1, b2 = bufs[0]
    t = GTAIL0
    g = base + t
    pltpu.async_copy(p_hbm.at[isl1.at[pl.ds(t, GTAIL)]], b1.at[pl.ds(0, GTAIL)], sga).wait()
    pltpu.async_copy(q_hbm.at[isl2.at[pl.ds(t, GTAIL)]], b2.at[pl.ds(0, GTAIL)], sga).wait()
    pltpu.sync_copy(b1.at[pl.ds(0, GTAIL)], hi_hbm.at[pl.ds(g, GTAIL)])
    pltpu.sync_copy(b2.at[pl.ds(0, GTAIL)], hj_hbm.at[pl.ds(g, GTAIL)])


def _sc_gather(p, q, src, dst):
    mesh = plsc.VectorSubcoreMesh(core_axis_name="c", subcore_axis_name="s")
    f = pl.kernel(
        _sc_gather_body,
        mesh=mesh,
        out_type=[jax.ShapeDtypeStruct((E, H), jnp.float32)] * 2,
        scratch_types=[
            pltpu.VMEM((PER_W,), jnp.int32),
            pltpu.VMEM((PER_W,), jnp.int32),
            pltpu.VMEM((CHG, H), jnp.float32),
            pltpu.VMEM((CHG, H), jnp.float32),
            pltpu.VMEM((CHG, H), jnp.float32),
            pltpu.VMEM((CHG, H), jnp.float32),
            pltpu.SemaphoreType.DMA,
            pltpu.SemaphoreType.DMA,
            pltpu.SemaphoreType.DMA,
            pltpu.SemaphoreType.DMA,
        ],
    )
    return f(p, q, src, dst)


def _edge_body(hi_ref, hj_ref, e2g_ref, l_ref, fd_ref, wd_ref, w2_ref, b2_ref,
               ef_ref):
    f = fd_ref[...]
    g_col = e2g_ref[0]
    onehot = (lax.broadcasted_iota(jnp.int32, (g_col.shape[0], G), 1)
              == g_col).astype(jnp.float32)
    lp = jnp.dot(onehot, l_ref[...], precision=_P)
    pre = hi_ref[...] + hj_ref[...] + lp
    for k in range(3):
        pre = pre + f[:, k:k + 1] * wd_ref[k:k + 1, :]
    h = _silu(pre)
    ef_ref[...] = _silu(jnp.dot(h, w2_ref[...], precision=_P) + b2_ref[...])


def _edge(hi, hj, e2g3, lfull, fd, wd, w2, b2):
    be = 2000
    return pl.pallas_call(
        _edge_body,
        grid=(E // be,),
        in_specs=[
            pl.BlockSpec((be, H), lambda i: (i, 0)),
            pl.BlockSpec((be, H), lambda i: (i, 0)),
            pl.BlockSpec((1, be, 1), lambda i: (i, 0, 0)),
            pl.BlockSpec((G, H), lambda i: (0, 0)),
            pl.BlockSpec((be, 3), lambda i: (i, 0)),
            pl.BlockSpec((3, H), lambda i: (0, 0)),
            pl.BlockSpec((H, H), lambda i: (0, 0)),
            pl.BlockSpec((1, H), lambda i: (0, 0)),
        ],
        out_specs=pl.BlockSpec((be, H), lambda i: (i, 0)),
        out_shape=jax.ShapeDtypeStruct((E, H), jnp.float32),
    )(hi, hj, e2g3, lfull, fd, wd, w2, b2)


def _sc_scatter_body(ef_hbm, dst_hbm, z_hbm, ones_hbm,
                     sums_hbm, cntb_hbm,
                     efbuf, idxbuf, onesbuf, ssum):
    cid = lax.axis_index("c")
    sid = lax.axis_index("s")
    row0 = sid * ROWS_A
    pltpu.sync_copy(z_hbm.at[pl.ds(row0, ROWS_A)],
                    ssum.at[pl.ds(row0, ROWS_A)])

    @pl.when(sid == 0)
    def _():
        pltpu.sync_copy(z_hbm.at[pl.ds(TAIL0, TAIL)],
                        ssum.at[pl.ds(TAIL0, TAIL)])

    pltpu.sync_copy(ones_hbm, onesbuf)
    plsc.subcore_barrier()

    base = sid * PER_S

    @pl.loop(0, PER_S, step=CH)
    def _(off):
        b = base + off
        pltpu.sync_copy(dst_hbm.at[pl.ds(b, CH)], idxbuf)

        @pl.when(cid == 0)
        def _():
            pltpu.sync_copy(ef_hbm.at[pl.ds(b, CH)], efbuf)
            pltpu.sync_copy(efbuf, ssum.at[idxbuf], add=True)

        @pl.when(cid == 1)
        def _():
            pltpu.sync_copy(onesbuf, ssum.at[idxbuf], add=True)

    plsc.subcore_barrier()

    @pl.when(cid == 0)
    def _():
        pltpu.sync_copy(ssum.at[pl.ds(row0, ROWS_A)],
                        sums_hbm.at[pl.ds(row0, ROWS_A)])

        @pl.when(sid == 0)
        def _():
            pltpu.sync_copy(ssum.at[pl.ds(TAIL0, TAIL)],
                            sums_hbm.at[pl.ds(TAIL0, TAIL)])

    @pl.when(cid == 1)
    def _():
        pltpu.sync_copy(ssum.at[pl.ds(row0, ROWS_A)],
                        cntb_hbm.at[pl.ds(row0, ROWS_A)])

        @pl.when(sid == 0)
        def _():
            pltpu.sync_copy(ssum.at[pl.ds(TAIL0, TAIL)],
                            cntb_hbm.at[pl.ds(TAIL0, TAIL)])


def _sc_scatter(ef, dst, zeros_n, ones_ch):
    mesh = plsc.VectorSubcoreMesh(core_axis_name="c", subcore_axis_name="s")
    f = pl.kernel(
        _sc_scatter_body,
        mesh=mesh,
        out_type=[
            jax.ShapeDtypeStruct((N, H), jnp.float32),
            jax.ShapeDtypeStruct((N, H), jnp.float32),
        ],
        scratch_types=[
            pltpu.VMEM((CH, H), jnp.float32),
            pltpu.VMEM((CH,), jnp.int32),
            pltpu.VMEM((CH, H), jnp.float32),
            pltpu.VMEM_SHARED((N, H), jnp.float32),
        ],
    )
    return f(ef, dst, zeros_n, ones_ch)


def _node_body(nf_ref, sp_ref, cp_ref, wa_ref, wb_ref, b1_ref, w2_ref, b2_ref,
               y_ref):
    nf = nf_ref[...]
    s = sp_ref[...]
    c = cp_ref[:, 0:1]
    agg = s / jnp.maximum(c, 1.0)
    o1 = _silu(jnp.dot(nf, wa_ref[...], precision=_P)
               + jnp.dot(agg, wb_ref[...], precision=_P) + b1_ref[...])
    y_ref[...] = nf + _silu(jnp.dot(o1, w2_ref[...], precision=_P) + b2_ref[...])


def _node(nf, sums, cnts, nw1a, nw1b, nb1, nw2, nb2):
    bn = 2000
    return pl.pallas_call(
        _node_body,
        grid=(N // bn,),
        in_specs=[
            pl.BlockSpec((bn, H), lambda i: (i, 0)),
            pl.BlockSpec((bn, H), lambda i: (i, 0)),
            pl.BlockSpec((bn, H), lambda i: (i, 0)),
            pl.BlockSpec((H, H), lambda i: (0, 0)),
            pl.BlockSpec((H, H), lambda i: (0, 0)),
            pl.BlockSpec((1, H), lambda i: (0, 0)),
            pl.BlockSpec((H, H), lambda i: (0, 0)),
            pl.BlockSpec((1, H), lambda i: (0, 0)),
        ],
        out_specs=pl.BlockSpec((bn, H), lambda i: (i, 0)),
        out_shape=jax.ShapeDtypeStruct((N, H), jnp.float32),
    )(nf, sums, cnts, nw1a, nw1b, nb1, nw2, nb2)


def kernel(node_features, frac_coords, lattices, edge_index, edge2graph,
           frac_diff, ew1, eb1, ew2, eb2, nw1, nb1, nw2, nb2):
    del frac_coords
    lat9 = lattices.reshape(G, 9)
    src = edge_index[0]
    dst = edge_index[1]

    p, q = _prep(node_features, ew1[:H], ew1[H:2 * H])
    lfull = _lat(lat9, ew1[2 * H:2 * H + 9], eb1.reshape(1, H))
    hi, hj = _sc_gather(p, q, src, dst)
    e2g3 = edge2graph.reshape(E // 2000, 2000, 1)
    ef = _edge(hi, hj, e2g3, lfull, frac_diff, ew1[2 * H + 9:], ew2,
               eb2.reshape(1, H))
    sums, cnts = _sc_scatter(ef, dst,
                             jnp.zeros((N, H), jnp.float32),
                             jnp.ones((CH, H), jnp.float32))
    y = _node(node_features, sums, cnts, nw1[:H], nw1[H:], nb1.reshape(1, H),
              nw2, nb2.reshape(1, H))
    return (y, ef)
